# baseline (device time: 27910 ns/iter reference)
import jax
import jax.numpy as jnp
from jax import lax
from jax.experimental import pallas as pl
from jax.experimental.pallas import tpu as pltpu

N_DEV = 4


def kernel(A, B):
    m, k = A.shape
    _, n = B.shape
    half = m // 2
    sub = m // 4
    qtr = m // 8

    f32 = jnp.float32
    bf16 = jnp.bfloat16

    def body(a_ref, b_ref, out_ref,
             acc, s1, s2, s3, r1, r2, r3, send_sems, recv_sems,
             copy_sems):
        my = lax.axis_index("i")
        a_bit = my % 2
        b_bit = my // 2
        pa = my + 1 - 2 * a_bit
        pb = 3 - my

        ka = (a_bit + b_bit) % 2
        ka2 = b_bit

        send_a = (1 - ka) * sub
        keep_a = ka * sub
        send_b = half + (1 - ka2) * sub
        keep_b = half + ka2 * sub

        def rc(src, dst, si, hf, dev):
            return pltpu.make_async_remote_copy(
                src_ref=src, dst_ref=dst,
                send_sem=send_sems.at[si, hf],
                recv_sem=recv_sems.at[si, hf],
                device_id=(dev,), device_id_type=pl.DeviceIdType.MESH,
            )

        def up(x):
            return x.astype(f32)

        barrier_sem = pltpu.get_barrier_semaphore()
        for nbr in [pa, pb]:
            pl.semaphore_signal(
                barrier_sem, inc=1,
                device_id=(nbr,), device_id_type=pl.DeviceIdType.MESH,
            )
        pl.semaphore_wait(barrier_sem, 2)

        def mm_bf16(rows, dst):
            dst[:, :] = jnp.dot(a_ref[pl.ds(rows, qtr), :], b_ref[:, :],
                                preferred_element_type=f32).astype(bf16)

        def mm_f32(rows):
            acc[pl.ds(rows, qtr), :] = jnp.dot(
                a_ref[pl.ds(rows, qtr), :], b_ref[:, :],
                preferred_element_type=f32)

        def out_copy(rows, ci, hf, p):
            c = pltpu.make_async_copy(
                acc.at[pl.ds(rows, qtr), :],
                out_ref.at[pl.ds(rows, qtr), :],
                copy_sems.at[ci, hf, p],
            )
            c.start()
            return c

        copies = []
        p1 = [[None, None], [None, None]]
        for p in range(2):
            mm_bf16(send_a + p * qtr, s1.at[0, p])
            p1[0][p] = rc(s1.at[0, p], r1.at[0, p], 0 + p, 0, pa)
            p1[0][p].start()
            mm_bf16(send_b + p * qtr, s1.at[1, p])
            p1[1][p] = rc(s1.at[1, p], r1.at[1, p], 0 + p, 1, pb)
            p1[1][p].start()

        mm_f32(keep_a)
        mm_f32(keep_b)
        mm_f32(keep_a + qtr)
        mm_f32(keep_b + qtr)

        p2 = [[None, None], [None, None]]
        keeps = [keep_a, keep_b]
        partner2 = [pb, pa]
        for p in range(2):
            for hf in range(2):
                rows = keeps[hf] + p * qtr
                p1[hf][p].wait_recv()
                s2[hf, p, :, :] = (
                    acc[pl.ds(rows, qtr), :] + up(r1[hf, p, :, :])
                ).astype(bf16)
                p2[hf][p] = rc(s2.at[hf, p], r2.at[hf, p],
                               2 + p, hf, partner2[hf])
                p2[hf][p].start()
                acc[pl.ds(rows, qtr), :] = (
                    acc[pl.ds(rows, qtr), :] + up(r1[hf, p, :, :])
                )

        p3 = [[None, None], [None, None]]
        partner3 = [pa, pb]
        for p in range(2):
            for hf in range(2):
                rows = keeps[hf] + p * qtr
                p2[hf][p].wait_recv()
                acc[pl.ds(rows, qtr), :] = (
                    acc[pl.ds(rows, qtr), :] + up(r2[hf, p, :, :])
                )
                s3[hf, p, :, :] = acc[pl.ds(rows, qtr), :].astype(bf16)
                p3[hf][p] = rc(s3.at[hf, p], r3.at[hf, p],
                               4 + p, hf, partner3[hf])
                p3[hf][p].start()
                copies.append(out_copy(rows, 0, hf, p))

        sends = [send_a, send_b]
        for p in range(2):
            for hf in range(2):
                p3[hf][p].wait_recv()
                rows = sends[hf] + p * qtr
                acc[pl.ds(rows, qtr), :] = up(r3[hf, p, :, :])
                copies.append(out_copy(rows, 1, hf, p))

        for group in [p1, p2, p3]:
            for hf in range(2):
                for p in range(2):
                    group[hf][p].wait_send()
        for c in copies:
            c.wait()

    return pl.pallas_call(
        body,
        out_shape=jax.ShapeDtypeStruct((m, n), f32),
        in_specs=[
            pl.BlockSpec(memory_space=pltpu.VMEM),
            pl.BlockSpec(memory_space=pltpu.VMEM),
        ],
        out_specs=pl.BlockSpec(memory_space=pltpu.HBM),
        scratch_shapes=[
            pltpu.VMEM((m, n), f32),
            pltpu.VMEM((2, 2, qtr, n), bf16),
            pltpu.VMEM((2, 2, qtr, n), bf16),
            pltpu.VMEM((2, 2, qtr, n), bf16),
            pltpu.VMEM((2, 2, qtr, n), bf16),
            pltpu.VMEM((2, 2, qtr, n), bf16),
            pltpu.VMEM((2, 2, qtr, n), bf16),
            pltpu.SemaphoreType.DMA((6, 2)),
            pltpu.SemaphoreType.DMA((6, 2)),
            pltpu.SemaphoreType.DMA((2, 2, 2)),
        ],
        compiler_params=pltpu.CompilerParams(collective_id=0),
    )(A, B)


# device time: 27862 ns/iter; 1.0017x vs baseline; 1.0017x over previous
import jax
import jax.numpy as jnp
from jax import lax
from jax.experimental import pallas as pl
from jax.experimental.pallas import tpu as pltpu

N_DEV = 4


def kernel(A, B):
    m, k = A.shape
    _, n = B.shape
    half = m // 2
    sub = m // 4
    qtr = m // 8

    f32 = jnp.float32
    bf16 = jnp.bfloat16

    def body(a_ref, b_ref, out_ref,
             acc, s1, s2, s3, r1, r2, r3, send_sems, recv_sems,
             copy_sems):
        my = lax.axis_index("i")
        a_bit = my % 2
        b_bit = my // 2
        pa = my + 1 - 2 * a_bit
        pb = 3 - my

        ka = (a_bit + b_bit) % 2
        ka2 = b_bit

        send_a = (1 - ka) * sub
        keep_a = ka * sub
        send_b = half + (1 - ka2) * sub
        keep_b = half + ka2 * sub

        def rc(src, dst, si, hf, dev):
            return pltpu.make_async_remote_copy(
                src_ref=src, dst_ref=dst,
                send_sem=send_sems.at[si, hf],
                recv_sem=recv_sems.at[si, hf],
                device_id=(dev,), device_id_type=pl.DeviceIdType.MESH,
            )

        def up(x):
            return x.astype(f32)

        barrier_sem = pltpu.get_barrier_semaphore()
        for nbr in [pa, pb]:
            pl.semaphore_signal(
                barrier_sem, inc=1,
                device_id=(nbr,), device_id_type=pl.DeviceIdType.MESH,
            )
        pl.semaphore_wait(barrier_sem, 2)

        def mm_bf16(rows, dst):
            dst[:, :] = jnp.dot(a_ref[pl.ds(rows, qtr), :], b_ref[:, :],
                                preferred_element_type=f32).astype(bf16)

        def mm_f32(rows):
            acc[pl.ds(rows, qtr), :] = jnp.dot(
                a_ref[pl.ds(rows, qtr), :], b_ref[:, :],
                preferred_element_type=f32)

        def out_copy(rows, ci, hf, p):
            c = pltpu.make_async_copy(
                acc.at[pl.ds(rows, qtr), :],
                out_ref.at[pl.ds(rows, qtr), :],
                copy_sems.at[ci, hf, p],
            )
            c.start()
            return c

        copies = []
        p1 = [[None, None], [None, None]]
        for p in range(2):
            mm_bf16(send_a + p * qtr, s1.at[0, p])
            p1[0][p] = rc(s1.at[0, p], r1.at[0, p], 0 + p, 0, pa)
            p1[0][p].start()
            mm_bf16(send_b + p * qtr, s1.at[1, p])
            p1[1][p] = rc(s1.at[1, p], r1.at[1, p], 0 + p, 1, pb)
            p1[1][p].start()

        mm_f32(keep_a)
        mm_f32(keep_b)
        mm_f32(keep_a + qtr)
        mm_f32(keep_b + qtr)

        p2 = [[None, None], [None, None]]
        keeps = [keep_a, keep_b]
        partner2 = [pb, pa]
        for p in range(2):
            for hf in range(2):
                rows = keeps[hf] + p * qtr
                p1[hf][p].wait_recv()
                t = acc[pl.ds(rows, qtr), :] + up(r1[hf, p, :, :])
                s2[hf, p, :, :] = t.astype(bf16)
                p2[hf][p] = rc(s2.at[hf, p], r2.at[hf, p],
                               2 + p, hf, partner2[hf])
                p2[hf][p].start()
                acc[pl.ds(rows, qtr), :] = t

        p3 = [[None, None], [None, None]]
        partner3 = [pa, pb]
        for p in range(2):
            for hf in range(2):
                rows = keeps[hf] + p * qtr
                p2[hf][p].wait_recv()
                t = acc[pl.ds(rows, qtr), :] + up(r2[hf, p, :, :])
                acc[pl.ds(rows, qtr), :] = t
                s3[hf, p, :, :] = t.astype(bf16)
                p3[hf][p] = rc(s3.at[hf, p], r3.at[hf, p],
                               4 + p, hf, partner3[hf])
                p3[hf][p].start()
                copies.append(out_copy(rows, 0, hf, p))

        sends = [send_a, send_b]
        for p in range(2):
            for hf in range(2):
                p3[hf][p].wait_recv()
                rows = sends[hf] + p * qtr
                acc[pl.ds(rows, qtr), :] = up(r3[hf, p, :, :])
                copies.append(out_copy(rows, 1, hf, p))

        for group in [p1, p2, p3]:
            for hf in range(2):
                for p in range(2):
                    group[hf][p].wait_send()
        for c in copies:
            c.wait()

    return pl.pallas_call(
        body,
        out_shape=jax.ShapeDtypeStruct((m, n), f32),
        in_specs=[
            pl.BlockSpec(memory_space=pltpu.VMEM),
            pl.BlockSpec(memory_space=pltpu.VMEM),
        ],
        out_specs=pl.BlockSpec(memory_space=pltpu.HBM),
        scratch_shapes=[
            pltpu.VMEM((m, n), f32),
            pltpu.VMEM((2, 2, qtr, n), bf16),
            pltpu.VMEM((2, 2, qtr, n), bf16),
            pltpu.VMEM((2, 2, qtr, n), bf16),
            pltpu.VMEM((2, 2, qtr, n), bf16),
            pltpu.VMEM((2, 2, qtr, n), bf16),
            pltpu.VMEM((2, 2, qtr, n), bf16),
            pltpu.SemaphoreType.DMA((6, 2)),
            pltpu.SemaphoreType.DMA((6, 2)),
            pltpu.SemaphoreType.DMA((2, 2, 2)),
        ],
        compiler_params=pltpu.CompilerParams(collective_id=0),
    )(A, B)
